# baseline (speedup 1.0000x reference)
import functools

import jax
import jax.numpy as jnp
from jax import lax
from jax.experimental import pallas as pl
from jax.experimental.pallas import tpu as pltpu
from jax.experimental.pallas import tpu_sc as plsc

B = 16384
N_CAT = 26
N_NUM = 13
VOCAB = 100000
EMB = 16
OUT = 128

NC = 2
NS = 16
NW = NC * NS
LOOKUPS = B * N_CAT
PER_W = LOOKUPS // NW
IDX_ROWS = PER_W // 128
CHUNK_ROWS = 13
N_CHUNKS = IDX_ROWS // CHUNK_ROWS
CHUNK = CHUNK_ROWS * 128


F_SC = N_CAT // 2
ROWS_SC = F_SC * VOCAB
WC = 800
SLABS_SC = ROWS_SC // WC
SLABS_T = -(-SLABS_SC // NS)
LK_SC = LOOKUPS // 2


def _sc_gather(tabT, idxT2d):
    mesh = plsc.VectorSubcoreMesh(core_axis_name="c", subcore_axis_name="s")

    @functools.partial(
        pl.kernel,
        mesh=mesh,
        out_type=(
            jax.ShapeDtypeStruct((LOOKUPS, EMB), jnp.float32),
            jax.ShapeDtypeStruct((N_CAT * VOCAB, EMB), jnp.float32),
        ),
        scratch_types=[
            pltpu.VMEM((EMB, WC), jnp.float32),
            pltpu.VMEM((WC, EMB), jnp.float32),
            pltpu.VMEM((IDX_ROWS, 128), jnp.int32),
            pltpu.VMEM((IDX_ROWS, 128), jnp.int32),
            pltpu.VMEM((CHUNK, EMB), jnp.float32),
            pltpu.VMEM((CHUNK, EMB), jnp.float32),
            pltpu.SemaphoreType.DMA,
            pltpu.SemaphoreType.DMA,
        ],
        compiler_params=pltpu.CompilerParams(
            use_tc_tiling_on_sc=False, needs_layout_passes=False),
    )
    def k(tab_hbm, idx_hbm, out_hbm, tlin_hbm,
          a_v, b_v, idx_v, dst_v, rows0, rows1, gsem, ssem):
        sc = lax.axis_index("c")
        tile = lax.axis_index("s")
        lane = lax.iota(jnp.int32, 16)

        def slab(ki, carry):
            s = tile + ki * NS

            @pl.when(s < SLABS_SC)
            def _():
                srow = sc * ROWS_SC + s * WC
                f = srow // VOCAB
                i0 = srow - f * VOCAB
                pltpu.sync_copy(
                    tab_hbm.at[pl.ds(f * EMB, EMB), pl.ds(i0, WC)], a_v)

                def col(i, c2):
                    b_v[i] = plsc.load_gather(a_v, [lane, i + lane * 0])
                    return c2

                lax.fori_loop(0, WC, col, 0)
                pltpu.sync_copy(b_v, tlin_hbm.at[pl.ds(srow, WC)])

            return carry

        lax.fori_loop(0, SLABS_T, slab, 0)
        plsc.subcore_barrier()

        wstart = sc * LK_SC + tile * PER_W
        pltpu.sync_copy(
            idx_hbm.at[pl.ds(wstart // 128, IDX_ROWS)], idx_v)
        lane26 = lane * N_CAT

        def build(s, carry):
            gpos = wstart + s * 128
            f = gpos // B
            b0 = gpos - f * B
            off = f * VOCAB
            for lb in range(8):
                sl = (s, pl.ds(lb * 16, 16))
                idx_v[sl] = idx_v[sl] + off
                dst_v[sl] = (b0 + lb * 16) * N_CAT + f + lane26
            return carry

        lax.fori_loop(0, IDX_ROWS, build, 0)

        bufs = (rows0, rows1)
        pending = [None, None]
        for c in range(N_CHUNKS):
            buf = bufs[c % 2]
            if pending[c % 2] is not None:
                for d in pending[c % 2]:
                    d.wait()
            gathers = []
            for j in range(CHUNK_ROWS):
                s = c * CHUNK_ROWS + j
                gathers.append(pltpu.async_copy(
                    tlin_hbm.at[idx_v.at[s]],
                    buf.at[pl.ds(j * 128, 128)],
                    gsem,
                ))
            for d in gathers:
                d.wait()
            scatters = []
            for j in range(CHUNK_ROWS):
                s = c * CHUNK_ROWS + j
                scatters.append(pltpu.async_copy(
                    buf.at[pl.ds(j * 128, 128)],
                    out_hbm.at[dst_v.at[s]],
                    ssem,
                ))
            pending[c % 2] = scatters
        for p in pending:
            if p is not None:
                for d in p:
                    d.wait()

    return k(tabT, idxT2d)[0]


IN_E = N_CAT * EMB
BLK = 2048
G = B // BLK


def _tc_tail_body(num_ref, emb_ref, gn, ge, bn, be, w1, w2, bb,
                  out_ref, sn, sqn, se, sqe):
    p = pl.program_id(0)
    i = pl.program_id(1)

    @pl.when(p == 0)
    def _stats():
        nblk = num_ref[...]
        eblk = emb_ref[...]
        s1 = jnp.sum(nblk, axis=0, keepdims=True)
        q1 = jnp.sum(nblk * nblk, axis=0, keepdims=True)
        s2 = jnp.sum(eblk, axis=0, keepdims=True)
        q2 = jnp.sum(eblk * eblk, axis=0, keepdims=True)

        @pl.when(i == 0)
        def _():
            sn[...] = s1
            sqn[...] = q1
            se[...] = s2
            sqe[...] = q2

        @pl.when(i > 0)
        def _():
            sn[...] += s1
            sqn[...] += q1
            se[...] += s2
            sqe[...] += q2

        @pl.when(i == G - 1)
        def _():
            inv_b = 1.0 / B
            mn = sn[...] * inv_b
            vn = sqn[...] * inv_b - mn * mn
            scale_n = gn[...] * lax.rsqrt(vn + 1e-5)
            sn[...] = scale_n
            sqn[...] = bn[...] - mn * scale_n
            me = se[...] * inv_b
            ve = sqe[...] * inv_b - me * me
            scale_e = ge[...] * lax.rsqrt(ve + 1e-5)
            se[...] = scale_e
            sqe[...] = be[...] - me * scale_e

    @pl.when(p == 1)
    def _matmul():
        h_n = jnp.maximum(num_ref[...] * sn[...] + sqn[...], 0.0)
        h_e = jnp.maximum(emb_ref[...] * se[...] + sqe[...], 0.0)
        dn = (((1,), (1,)), ((), ()))
        out_ref[...] = (
            lax.dot_general(h_n, w1[...], dn,
                            preferred_element_type=jnp.float32,
                            precision=lax.Precision.HIGHEST)
            + lax.dot_general(h_e, w2[...], dn,
                              preferred_element_type=jnp.float32,
                              precision=lax.Precision.HIGHEST)
            + bb[...]
        )


def _tc_tail(num, emb, gn, ge, bn, be, w1, w2, bb):
    full = lambda shape: pl.BlockSpec(shape, lambda p, i: (0, 0))
    blk = lambda shape: pl.BlockSpec(shape, lambda p, i: (i, 0))
    return pl.pallas_call(
        _tc_tail_body,
        grid=(2, G),
        in_specs=[
            blk((BLK, N_NUM)),
            blk((BLK, IN_E)),
            full((1, N_NUM)),
            full((1, IN_E)),
            full((1, N_NUM)),
            full((1, IN_E)),
            full((OUT, N_NUM)),
            full((OUT, IN_E)),
            full((1, OUT)),
        ],
        out_specs=blk((BLK, OUT)),
        out_shape=jax.ShapeDtypeStruct((B, OUT), jnp.float32),
        scratch_shapes=[
            pltpu.VMEM((1, N_NUM), jnp.float32),
            pltpu.VMEM((1, N_NUM), jnp.float32),
            pltpu.VMEM((1, IN_E), jnp.float32),
            pltpu.VMEM((1, IN_E), jnp.float32),
        ],
    )(num, emb, gn, ge, bn, be, w1, w2, bb)


def kernel(num, cat_indices, tables, gamma, beta, W, b):
    tabT = tables.transpose(0, 2, 1).reshape(N_CAT * EMB, VOCAB)
    idxT2d = cat_indices.T.reshape(LOOKUPS // 128, 128)
    emb_flat = _sc_gather(tabT, idxT2d)
    emb = emb_flat.reshape(B, IN_E)
    out = _tc_tail(
        num, emb,
        gamma[:N_NUM].reshape(1, N_NUM), gamma[N_NUM:].reshape(1, IN_E),
        beta[:N_NUM].reshape(1, N_NUM), beta[N_NUM:].reshape(1, IN_E),
        W[:, :N_NUM], W[:, N_NUM:],
        b.reshape(1, OUT),
    )
    return out


# speedup vs baseline: 3.0033x; 3.0033x over previous
import functools

import jax
import jax.numpy as jnp
from jax import lax
from jax.experimental import pallas as pl
from jax.experimental.pallas import tpu as pltpu
from jax.experimental.pallas import tpu_sc as plsc

B = 16384
N_CAT = 26
N_NUM = 13
VOCAB = 100000
EMB = 16
OUT = 128
IN_E = N_CAT * EMB

NC = 2
NS = 16
NW = NC * NS
PLANES = N_CAT * EMB
P_T = PLANES // NW
OCHUNK = 4096


def _sc_gather_t(tabT, catT):
    mesh = plsc.VectorSubcoreMesh(core_axis_name="c", subcore_axis_name="s")

    @functools.partial(
        pl.kernel,
        mesh=mesh,
        out_type=jax.ShapeDtypeStruct((PLANES, B), jnp.float32),
        scratch_types=[
            pltpu.VMEM((VOCAB,), jnp.float32),
            pltpu.VMEM((B,), jnp.int32),
            pltpu.VMEM((OCHUNK,), jnp.float32),
        ],
        compiler_params=pltpu.CompilerParams(
            use_tc_tiling_on_sc=False, needs_layout_passes=False),
    )
    def k(tab_hbm, idx_hbm, out_hbm, p_v, ix_v, o_v):
        wid = lax.axis_index("c") * NS + lax.axis_index("s")
        for j in range(P_T):
            r = wid * P_T + j
            f = r // EMB
            pltpu.sync_copy(tab_hbm.at[r], p_v)
            pltpu.sync_copy(idx_hbm.at[f], ix_v)
            for cc in range(B // OCHUNK):
                def gath(k2, carry, _cc=cc):
                    o_v[pl.ds(k2 * 16, 16)] = plsc.load_gather(
                        p_v, [ix_v[pl.ds(_cc * OCHUNK + k2 * 16, 16)]])
                    return carry

                lax.fori_loop(0, OCHUNK // 16, gath, 0)
                pltpu.sync_copy(
                    o_v, out_hbm.at[r, pl.ds(cc * OCHUNK, OCHUNK)])

    return k(tabT, catT)


BLK = 2048
G = B // BLK


def _tc_tail_body(num_ref, embt_ref, gn, ge, bn, be, w1, w2, bb,
                  out_ref, sn, sqn, se, sqe):
    p = pl.program_id(0)
    i = pl.program_id(1)

    @pl.when(p == 0)
    def _stats():
        nblk = num_ref[...]
        eblk = embt_ref[...]
        s1 = jnp.sum(nblk, axis=0, keepdims=True)
        q1 = jnp.sum(nblk * nblk, axis=0, keepdims=True)
        s2 = jnp.sum(eblk, axis=1, keepdims=True)
        q2 = jnp.sum(eblk * eblk, axis=1, keepdims=True)

        @pl.when(i == 0)
        def _():
            sn[...] = s1
            sqn[...] = q1
            se[...] = s2
            sqe[...] = q2

        @pl.when(i > 0)
        def _():
            sn[...] += s1
            sqn[...] += q1
            se[...] += s2
            sqe[...] += q2

        @pl.when(i == G - 1)
        def _():
            inv_b = 1.0 / B
            mn = sn[...] * inv_b
            vn = sqn[...] * inv_b - mn * mn
            scale_n = gn[...] * lax.rsqrt(vn + 1e-5)
            sn[...] = scale_n
            sqn[...] = bn[...] - mn * scale_n
            me = se[...] * inv_b
            ve = sqe[...] * inv_b - me * me
            scale_e = ge[...] * lax.rsqrt(ve + 1e-5)
            se[...] = scale_e
            sqe[...] = be[...] - me * scale_e

    @pl.when(p == 1)
    def _matmul():
        h_n = jnp.maximum(num_ref[...] * sn[...] + sqn[...], 0.0)
        h_e = jnp.maximum(embt_ref[...] * se[...] + sqe[...], 0.0)
        out_ref[...] = (
            lax.dot_general(h_n, w1[...], (((1,), (1,)), ((), ())),
                            preferred_element_type=jnp.float32,
                            precision=lax.Precision.HIGHEST)
            + lax.dot_general(h_e, w2[...], (((0,), (1,)), ((), ())),
                              preferred_element_type=jnp.float32,
                              precision=lax.Precision.HIGHEST)
            + bb[...]
        )


def _tc_tail(num, embT, gn, ge, bn, be, w1, w2, bb):
    full = lambda shape: pl.BlockSpec(shape, lambda p, i: (0, 0))
    rowblk = lambda shape: pl.BlockSpec(shape, lambda p, i: (i, 0))
    colblk = lambda shape: pl.BlockSpec(shape, lambda p, i: (0, i))
    return pl.pallas_call(
        _tc_tail_body,
        grid=(2, G),
        in_specs=[
            rowblk((BLK, N_NUM)),
            colblk((IN_E, BLK)),
            full((1, N_NUM)),
            full((IN_E, 1)),
            full((1, N_NUM)),
            full((IN_E, 1)),
            full((OUT, N_NUM)),
            full((OUT, IN_E)),
            full((1, OUT)),
        ],
        out_specs=rowblk((BLK, OUT)),
        out_shape=jax.ShapeDtypeStruct((B, OUT), jnp.float32),
        scratch_shapes=[
            pltpu.VMEM((1, N_NUM), jnp.float32),
            pltpu.VMEM((1, N_NUM), jnp.float32),
            pltpu.VMEM((IN_E, 1), jnp.float32),
            pltpu.VMEM((IN_E, 1), jnp.float32),
        ],
    )(num, embT, gn, ge, bn, be, w1, w2, bb)


def kernel(num, cat_indices, tables, gamma, beta, W, b):
    tabT = tables.transpose(0, 2, 1).reshape(PLANES, VOCAB)
    catT = cat_indices.T
    embT = _sc_gather_t(tabT, catT)
    out = _tc_tail(
        num, embT,
        gamma[:N_NUM].reshape(1, N_NUM), gamma[N_NUM:].reshape(IN_E, 1),
        beta[:N_NUM].reshape(1, N_NUM), beta[N_NUM:].reshape(IN_E, 1),
        W[:, :N_NUM], W[:, N_NUM:],
        b.reshape(1, OUT),
    )
    return out


# speedup vs baseline: 3.2798x; 1.0921x over previous
import functools

import jax
import jax.numpy as jnp
from jax import lax
from jax.experimental import pallas as pl
from jax.experimental.pallas import tpu as pltpu
from jax.experimental.pallas import tpu_sc as plsc

B = 16384
N_CAT = 26
N_NUM = 13
VOCAB = 100000
EMB = 16
OUT = 128
IN_E = N_CAT * EMB

NC = 2
NS = 16
NW = NC * NS
PLANES = N_CAT * EMB
P_T = PLANES // NW
OCHUNK = 4096


UNROLL = 4


def _sc_gather_t(tabT, catT):
    mesh = plsc.VectorSubcoreMesh(core_axis_name="c", subcore_axis_name="s")

    @functools.partial(
        pl.kernel,
        mesh=mesh,
        out_type=(
            jax.ShapeDtypeStruct((PLANES, B), jnp.float32),
            jax.ShapeDtypeStruct((PLANES, 16), jnp.float32),
        ),
        scratch_types=[
            pltpu.VMEM((VOCAB,), jnp.float32),
            pltpu.VMEM((B,), jnp.int32),
            pltpu.VMEM((OCHUNK,), jnp.float32),
            pltpu.VMEM((OCHUNK,), jnp.float32),
            pltpu.VMEM((16,), jnp.float32),
            pltpu.SemaphoreType.DMA,
            pltpu.SemaphoreType.DMA,
            pltpu.SemaphoreType.DMA,
        ],
        compiler_params=pltpu.CompilerParams(
            use_tc_tiling_on_sc=False, needs_layout_passes=False),
    )
    def k(tab_hbm, idx_hbm, out_hbm, st_hbm,
          p_v, ix_v, o_v0, o_v1, sv, psem, isem, osem):
        wid = lax.axis_index("c") * NS + lax.axis_index("s")
        lane = lax.iota(jnp.int32, 16)
        obufs = (o_v0, o_v1)
        pend = [None, None]
        for j in range(P_T):
            r = wid * P_T + j
            f = r // EMB
            dp = pltpu.async_copy(tab_hbm.at[r], p_v, psem)
            di = pltpu.async_copy(idx_hbm.at[f], ix_v, isem)
            dp.wait()
            di.wait()
            zero = jnp.zeros((16,), jnp.float32)
            sacc = zero
            qacc = zero
            for cc in range(B // OCHUNK):
                ob = obufs[cc % 2]
                if pend[cc % 2] is not None:
                    pend[cc % 2].wait()
                    pend[cc % 2] = None

                def gath(k2, carry, _cc=cc, _ob=ob):
                    sa, qa = carry
                    for u in range(UNROLL):
                        pos = k2 * (16 * UNROLL) + u * 16
                        v = plsc.load_gather(
                            p_v, [ix_v[pl.ds(_cc * OCHUNK + pos, 16)]])
                        _ob[pl.ds(pos, 16)] = v
                        sa = sa + v
                        qa = qa + v * v
                    return (sa, qa)

                sacc, qacc = lax.fori_loop(
                    0, OCHUNK // (16 * UNROLL), gath, (sacc, qacc))
                pend[cc % 2] = pltpu.async_copy(
                    ob, out_hbm.at[r, pl.ds(cc * OCHUNK, OCHUNK)], osem)
            ssum = jnp.sum(sacc)
            ssq = jnp.sum(qacc)
            sv[...] = jnp.where(lane == 0, ssum,
                                jnp.where(lane == 1, ssq, 0.0))
            pltpu.sync_copy(sv, st_hbm.at[r])
        for pd in pend:
            if pd is not None:
                pd.wait()

    return k(tabT, catT)


BLK = 2048
G = B // BLK


def _tc_tail_body(num_ref, embt_ref, st_ref, gn, ge, bn, be, w1, w2, bb,
                  out_ref, sn, sqn, se, sqe):
    p = pl.program_id(0)
    i = pl.program_id(1)

    @pl.when(p == 0)
    def _stats():
        nblk = num_ref[...]
        s1 = jnp.sum(nblk, axis=0, keepdims=True)
        q1 = jnp.sum(nblk * nblk, axis=0, keepdims=True)

        @pl.when(i == 0)
        def _():
            sn[...] = s1
            sqn[...] = q1

        @pl.when(i > 0)
        def _():
            sn[...] += s1
            sqn[...] += q1

        @pl.when(i == G - 1)
        def _():
            inv_b = 1.0 / B
            mn = sn[...] * inv_b
            vn = sqn[...] * inv_b - mn * mn
            scale_n = gn[...] * lax.rsqrt(vn + 1e-5)
            sn[...] = scale_n
            sqn[...] = bn[...] - mn * scale_n
            me = st_ref[:, 0:1] * inv_b
            ve = st_ref[:, 1:2] * inv_b - me * me
            scale_e = ge[...] * lax.rsqrt(ve + 1e-5)
            se[...] = scale_e
            sqe[...] = be[...] - me * scale_e

    @pl.when(p == 1)
    def _matmul():
        h_n = jnp.maximum(num_ref[...] * sn[...] + sqn[...], 0.0)
        h_e = jnp.maximum(embt_ref[...] * se[...] + sqe[...], 0.0)
        out_ref[...] = (
            lax.dot_general(h_n, w1[...], (((1,), (1,)), ((), ())),
                            preferred_element_type=jnp.float32,
                            precision=lax.Precision.HIGHEST)
            + lax.dot_general(h_e, w2[...], (((0,), (1,)), ((), ())),
                              preferred_element_type=jnp.float32,
                              precision=lax.Precision.HIGHEST)
            + bb[...]
        )


def _tc_tail(num, embT, stats, gn, ge, bn, be, w1, w2, bb):
    full = lambda shape: pl.BlockSpec(shape, lambda p, i: (0, 0))
    rowblk = lambda shape: pl.BlockSpec(shape, lambda p, i: (i, 0))
    colblk = lambda shape: pl.BlockSpec(shape, lambda p, i: (0, i * p))
    return pl.pallas_call(
        _tc_tail_body,
        grid=(2, G),
        in_specs=[
            rowblk((BLK, N_NUM)),
            colblk((IN_E, BLK)),
            full((IN_E, 16)),
            full((1, N_NUM)),
            full((IN_E, 1)),
            full((1, N_NUM)),
            full((IN_E, 1)),
            full((OUT, N_NUM)),
            full((OUT, IN_E)),
            full((1, OUT)),
        ],
        out_specs=rowblk((BLK, OUT)),
        out_shape=jax.ShapeDtypeStruct((B, OUT), jnp.float32),
        scratch_shapes=[
            pltpu.VMEM((1, N_NUM), jnp.float32),
            pltpu.VMEM((1, N_NUM), jnp.float32),
            pltpu.VMEM((IN_E, 1), jnp.float32),
            pltpu.VMEM((IN_E, 1), jnp.float32),
        ],
    )(num, embT, stats, gn, ge, bn, be, w1, w2, bb)


def kernel(num, cat_indices, tables, gamma, beta, W, b):
    tabT = tables.transpose(0, 2, 1).reshape(PLANES, VOCAB)
    catT = cat_indices.T
    embT, stats = _sc_gather_t(tabT, catT)
    out = _tc_tail(
        num, embT, stats,
        gamma[:N_NUM].reshape(1, N_NUM), gamma[N_NUM:].reshape(IN_E, 1),
        beta[:N_NUM].reshape(1, N_NUM), beta[N_NUM:].reshape(IN_E, 1),
        W[:, :N_NUM], W[:, N_NUM:],
        b.reshape(1, OUT),
    )
    return out
